# baseline (speedup 1.0000x reference)
import functools

import jax
import jax.numpy as jnp
from jax import lax
from jax.experimental import pallas as pl
from jax.experimental.pallas import tpu as pltpu
from jax.experimental.pallas import tpu_sc as plsc

V = 1000
T = 20
VP = 1024
L = 16
VPT = VP // 16

_info = plsc.get_sparse_core_info()
_NC, _NS = _info.num_cores, _info.num_subcores
NW = _NC * _NS


def _sc_body(seq, cols_per_w, cf_hbm, xt_hbm, out_hbm, cf_v, tbl_v, idx_v,
             res_v, shr_tbl, sem_i0, sem_i1, sem_c, sem_o):
    sub = lax.axis_index("s")
    wid = sub * _NC + lax.axis_index("c")
    c0 = wid * cols_per_w

    rsplit = 96
    h_idx0 = pltpu.async_copy(
        xt_hbm.at[pl.ds(0, rsplit), pl.ds(c0, cols_per_w)],
        idx_v.at[pl.ds(0, rsplit)], sem_i0)
    h_idx1 = pltpu.async_copy(
        xt_hbm.at[pl.ds(rsplit, seq - rsplit), pl.ds(c0, cols_per_w)],
        idx_v.at[pl.ds(rsplit, seq - rsplit)], sem_i1)

    base_v = jnp.minimum(sub * VPT, V - VPT)
    pltpu.async_copy(cf_hbm.at[pl.ds(base_v * T, VPT * T)], cf_v,
                     sem_c).wait()

    lanes = lax.iota(jnp.int32, L)

    @plsc.parallel_loop(0, VPT // L)
    def _chunk(j):
        col_idx = (lanes + j * L) * T
        best_v = plsc.load_gather(cf_v, [col_idx])
        best_i = jnp.zeros((L,), jnp.int32)
        for t in range(1, T):
            vals = plsc.load_gather(cf_v, [col_idx + t])
            m = vals > best_v
            best_v = jnp.where(m, vals, best_v)
            best_i = jnp.where(m, jnp.full((L,), t, jnp.int32), best_i)
        tbl_v[pl.ds(base_v + j * L, L)] = best_i

    pltpu.sync_copy(tbl_v.at[pl.ds(base_v, VPT)],
                    shr_tbl.at[pl.ds(base_v, VPT)])
    plsc.subcore_barrier()
    pltpu.sync_copy(shr_tbl, tbl_v)

    vecs = cols_per_w // L
    row_groups = [0, 48, rsplit, 144, seq]
    handles = []
    for g in range(len(row_groups) - 1):
        lo, hi = row_groups[g], row_groups[g + 1]
        if lo == 0:
            h_idx0.wait()
        elif lo == rsplit:
            h_idx1.wait()

        @plsc.parallel_loop(lo, hi, unroll=2)
        def _row(r):
            for u in range(vecs):
                idxs = idx_v[r, pl.ds(u * L, L)]
                res_v[r, pl.ds(u * L, L)] = plsc.load_gather(tbl_v, [idxs])

        handles.append(
            pltpu.async_copy(res_v.at[pl.ds(lo, hi - lo)],
                             out_hbm.at[pl.ds(lo, hi - lo),
                                        pl.ds(c0, cols_per_w)], sem_o))
    for h in handles:
        h.wait()


def kernel(x, x_chars, classifier):
    del x_chars
    batch, seq = x.shape
    cols_per_w = batch // NW
    cf = classifier.reshape(-1)

    k = functools.partial(
        pl.kernel,
        out_type=jax.ShapeDtypeStruct((seq, batch), jnp.int32),
        mesh=plsc.VectorSubcoreMesh(core_axis_name="c", subcore_axis_name="s"),
        compiler_params=pltpu.CompilerParams(
            needs_layout_passes=False, use_tc_tiling_on_sc=True),
        scratch_types=[
            pltpu.VMEM((VPT * T,), jnp.float32),
            pltpu.VMEM((VP,), jnp.int32),
            pltpu.VMEM((seq, cols_per_w), jnp.int32),
            pltpu.VMEM((seq, cols_per_w), jnp.int32),
            pltpu.VMEM_SHARED((VP,), jnp.int32),
            pltpu.SemaphoreType.DMA,
            pltpu.SemaphoreType.DMA,
            pltpu.SemaphoreType.DMA,
            pltpu.SemaphoreType.DMA,
        ],
    )(functools.partial(_sc_body, seq, cols_per_w))

    return k(cf, x.T).T


# speedup vs baseline: 1.0367x; 1.0367x over previous
import functools

import jax
import jax.numpy as jnp
from jax import lax
from jax.experimental import pallas as pl
from jax.experimental.pallas import tpu as pltpu
from jax.experimental.pallas import tpu_sc as plsc

V = 1000
T = 20
VP = 1024
L = 16
VPT = VP // 16

_info = plsc.get_sparse_core_info()
_NC, _NS = _info.num_cores, _info.num_subcores
NW = _NC * _NS


def _sc_body(seq, cols_per_w, cf_hbm, xt_hbm, out_hbm, cf_v, tbl_v, idx_v,
             res_v, shr_tbl, sem_i0, sem_i1, sem_c, sem_o):
    sub = lax.axis_index("s")
    wid = sub * _NC + lax.axis_index("c")
    c0 = wid * cols_per_w

    h_idx = pltpu.async_copy(xt_hbm.at[:, pl.ds(c0, cols_per_w)], idx_v,
                             sem_i0)
    del sem_i1

    base_v = jnp.minimum(sub * VPT, V - VPT)
    pltpu.async_copy(cf_hbm.at[pl.ds(base_v * T, VPT * T)], cf_v,
                     sem_c).wait()

    lanes = lax.iota(jnp.int32, L)

    @plsc.parallel_loop(0, VPT // L)
    def _chunk(j):
        col_idx = (lanes + j * L) * T
        best_v = plsc.load_gather(cf_v, [col_idx])
        best_i = jnp.zeros((L,), jnp.int32)
        for t in range(1, T):
            vals = plsc.load_gather(cf_v, [col_idx + t])
            m = vals > best_v
            best_v = jnp.where(m, vals, best_v)
            best_i = jnp.where(m, jnp.full((L,), t, jnp.int32), best_i)
        tbl_v[pl.ds(base_v + j * L, L)] = best_i

    pltpu.sync_copy(tbl_v.at[pl.ds(base_v, VPT)],
                    shr_tbl.at[pl.ds(base_v, VPT)])
    plsc.subcore_barrier()
    pltpu.sync_copy(shr_tbl, tbl_v)

    h_idx.wait()

    vecs = cols_per_w // L
    row_groups = [0, 48, 96, 144, seq]
    handles = []
    for g in range(len(row_groups) - 1):
        lo, hi = row_groups[g], row_groups[g + 1]

        @plsc.parallel_loop(lo, hi)
        def _row(r):
            for u in range(vecs):
                idxs = idx_v[r, pl.ds(u * L, L)]
                res_v[r, pl.ds(u * L, L)] = plsc.load_gather(tbl_v, [idxs])

        handles.append(
            pltpu.async_copy(res_v.at[pl.ds(lo, hi - lo)],
                             out_hbm.at[pl.ds(lo, hi - lo),
                                        pl.ds(c0, cols_per_w)], sem_o))
    for h in handles:
        h.wait()


def kernel(x, x_chars, classifier):
    del x_chars
    batch, seq = x.shape
    cols_per_w = batch // NW
    cf = classifier.reshape(-1)

    k = functools.partial(
        pl.kernel,
        out_type=jax.ShapeDtypeStruct((seq, batch), jnp.int32),
        mesh=plsc.VectorSubcoreMesh(core_axis_name="c", subcore_axis_name="s"),
        compiler_params=pltpu.CompilerParams(
            needs_layout_passes=False, use_tc_tiling_on_sc=True),
        scratch_types=[
            pltpu.VMEM((VPT * T,), jnp.float32),
            pltpu.VMEM((VP,), jnp.int32),
            pltpu.VMEM((seq, cols_per_w), jnp.int32),
            pltpu.VMEM((seq, cols_per_w), jnp.int32),
            pltpu.VMEM_SHARED((VP,), jnp.int32),
            pltpu.SemaphoreType.DMA,
            pltpu.SemaphoreType.DMA,
            pltpu.SemaphoreType.DMA,
            pltpu.SemaphoreType.DMA,
        ],
    )(functools.partial(_sc_body, seq, cols_per_w))

    return k(cf, x.T).T
